# baseline (speedup 1.0000x reference)
import functools

import jax
import jax.numpy as jnp
from jax.experimental import pallas as pl
from jax.experimental.pallas import tpu as pltpu
from jax.experimental.pallas import tpu_sc as plsc

_EPS = 1e-5
_LANES = 128


def _router_kernel(x_ref, wr_ref, br_ref, oi_ref, os_ref, *, E, M):
    lane = jax.lax.broadcasted_iota(jnp.int32, (M, _LANES), 1)
    xb = x_ref[...].astype(jnp.bfloat16)
    logits = jnp.dot(xb, wr_ref[...].astype(jnp.bfloat16),
                     preferred_element_type=jnp.float32) + br_ref[...]
    logits = jnp.where(lane < E, logits, -jnp.inf)
    big = jnp.int32(_LANES + 1)
    m1 = jnp.max(logits, axis=-1, keepdims=True)
    i1 = jnp.min(jnp.where(logits == m1, lane, big), axis=-1, keepdims=True)
    oh1 = lane == i1
    l2 = jnp.where(oh1, -jnp.inf, logits)
    m2 = jnp.max(l2, axis=-1, keepdims=True)
    i2 = jnp.min(jnp.where(l2 == m2, lane, big), axis=-1, keepdims=True)
    oh2 = lane == i2
    p = jnp.exp(logits - m1)
    p = jnp.where(lane < E, p, 0.0)
    p = p / jnp.sum(p, axis=-1, keepdims=True)
    s1 = jnp.sum(jnp.where(oh1, p, 0.0), axis=-1, keepdims=True)
    s2 = jnp.sum(jnp.where(oh2, p, 0.0), axis=-1, keepdims=True)
    oi_ref[...] = jnp.concatenate([i1, i2], axis=1)
    os_ref[...] = jnp.concatenate([s1, s2], axis=1)


def _router(xf, Wr, br, *, N, D, E, M):
    T = N // M
    wr_p = jnp.zeros((D, _LANES), Wr.dtype).at[:, :E].set(Wr)
    br_p = jnp.zeros((1, _LANES), br.dtype).at[0, :E].set(br)
    return pl.pallas_call(
        functools.partial(_router_kernel, E=E, M=M),
        grid=(T,),
        in_specs=[
            pl.BlockSpec((M, D), lambda t: (t, 0)),
            pl.BlockSpec((D, _LANES), lambda t: (0, 0)),
            pl.BlockSpec((1, _LANES), lambda t: (0, 0)),
        ],
        out_specs=[
            pl.BlockSpec((M, 2), lambda t: (t, 0)),
            pl.BlockSpec((M, 2), lambda t: (t, 0)),
        ],
        out_shape=[
            jax.ShapeDtypeStruct((N, 2), jnp.int32),
            jax.ShapeDtypeStruct((N, 2), jnp.float32),
        ],
    )(xf, wr_p, br_p)


def _meta_kernel(idx_ref, dest_ref, aend_ref, l_scr, rank_scr, cnt_scr,
                 ast_scr, *, E, Ms, Mal):
    d = pl.program_id(0)
    t = pl.program_id(1)
    lane = jax.lax.broadcasted_iota(jnp.int32, (Ms, _LANES), 1)
    eid = idx_ref[0]
    oh = lane == eid
    ohf = oh.astype(jnp.bfloat16)

    @pl.when(jnp.logical_and(d == 0, t == 0))
    def _init():
        r = jax.lax.broadcasted_iota(jnp.int32, (Ms, Ms), 0)
        c = jax.lax.broadcasted_iota(jnp.int32, (Ms, Ms), 1)
        l_scr[...] = (r > c).astype(jnp.bfloat16)
        cnt_scr[...] = jnp.zeros_like(cnt_scr)

    @pl.when(d == 0)
    def _pass_a():
        exc = jnp.dot(l_scr[...], ohf, preferred_element_type=jnp.float32)
        intra = jnp.sum(jnp.where(oh, exc, 0.0), axis=1, keepdims=True)
        base = jnp.sum(jnp.where(oh, cnt_scr[...], 0.0), axis=1, keepdims=True)
        rank_scr[pl.ds(t * Ms, Ms), :] = base + intra
        cnt_scr[...] += jnp.sum(ohf.astype(jnp.float32), axis=0, keepdims=True)

    @pl.when(jnp.logical_and(d == 1, t == 0))
    def _offsets():
        counts = cnt_scr[...]
        sizes = jnp.ceil(counts / Mal) * Mal
        r2 = jax.lax.broadcasted_iota(jnp.int32, (_LANES, _LANES), 0)
        c2 = jax.lax.broadcasted_iota(jnp.int32, (_LANES, _LANES), 1)
        lt = (r2 <= c2).astype(jnp.bfloat16)
        aend = jnp.dot(sizes.astype(jnp.bfloat16), lt,
                       preferred_element_type=jnp.float32)
        ast_scr[...] = aend - sizes
        aend_ref[...] = aend

    @pl.when(d == 1)
    def _pass_b():
        base = jnp.sum(jnp.where(oh, ast_scr[...], 0.0), axis=1, keepdims=True)
        dest = base + rank_scr[pl.ds(t * Ms, Ms), :]
        dest_ref[0] = dest.astype(jnp.int32)


def _dispatch_meta(idx2, *, N, E, K, Mal):
    S_tot = K * N
    Ms = 1024
    T = S_tot // Ms
    idx_r = idx2.reshape(T, Ms, 1)
    dest, aend = pl.pallas_call(
        functools.partial(_meta_kernel, E=E, Ms=Ms, Mal=Mal),
        grid=(2, T),
        in_specs=[pl.BlockSpec((1, Ms, 1), lambda d, t: (t, 0, 0))],
        out_specs=[
            pl.BlockSpec((1, Ms, 1), lambda d, t: (t, 0, 0)),
            pl.BlockSpec((1, _LANES), lambda d, t: (0, 0)),
        ],
        out_shape=[
            jax.ShapeDtypeStruct((T, Ms, 1), jnp.int32),
            jax.ShapeDtypeStruct((1, _LANES), jnp.float32),
        ],
        scratch_shapes=[
            pltpu.VMEM((Ms, Ms), jnp.bfloat16),
            pltpu.VMEM((S_tot, 1), jnp.float32),
            pltpu.VMEM((1, _LANES), jnp.float32),
            pltpu.VMEM((1, _LANES), jnp.float32),
        ],
    )(idx_r)
    return dest.reshape(S_tot), aend[0, :E]


_SC_MESH = None


def _sc_mesh():
    global _SC_MESH
    if _SC_MESH is None:
        _SC_MESH = plsc.VectorSubcoreMesh(core_axis_name="c",
                                          subcore_axis_name="s")
    return _SC_MESH


_SC_W = 128


def _sc_dispatch(xf, de, do, *, N, D, CAP, W=_SC_W):
    C = D // _LANES
    x8 = xf.reshape(N * C, _LANES)
    R = N * C

    @functools.partial(
        pl.kernel,
        out_type=jax.ShapeDtypeStruct((CAP * C, _LANES), jnp.float32),
        mesh=_sc_mesh(),
    )
    def k(x_hbm, ie_hbm, io_hbm, o_hbm):
        def body(x_vmem, ie_vmem, io_vmem):
            pltpu.sync_copy(x_vmem, o_hbm.at[ie_vmem.at[0]])
            pltpu.sync_copy(x_vmem, o_hbm.at[io_vmem.at[0]])

        pltpu.emit_pipeline(
            body,
            grid=(R // W,),
            in_specs=[
                pl.BlockSpec((W, _LANES), lambda i: (i, 0)),
                pl.BlockSpec((1, W), lambda i: (0, i)),
                pl.BlockSpec((1, W), lambda i: (0, i)),
            ],
            out_specs=[],
            core_axis_name=("c", "s"),
            dimension_semantics=(pltpu.PARALLEL,),
        )(x_hbm, ie_hbm, io_hbm)

    return k(x8, de, do).reshape(CAP, D)


def _sc_combine(ys, dest8, *, S_tot, D, W=_SC_W):
    C = D // _LANES
    y8 = ys.reshape(ys.shape[0] * C, _LANES)
    R = S_tot * C

    @functools.partial(
        pl.kernel,
        out_type=jax.ShapeDtypeStruct((R, _LANES), jnp.float32),
        mesh=_sc_mesh(),
    )
    def k(y_hbm, i_hbm, o_hbm):
        def body(i_vmem, o_vmem):
            pltpu.sync_copy(y_hbm.at[i_vmem.at[0]], o_vmem)

        pltpu.emit_pipeline(
            body,
            grid=(R // W,),
            in_specs=[pl.BlockSpec((1, W), lambda i: (0, i))],
            out_specs=[pl.BlockSpec((W, _LANES), lambda i: (i, 0))],
            core_axis_name=("c", "s"),
            dimension_semantics=(pltpu.PARALLEL,),
        )(i_hbm, o_hbm)

    return k(y8, dest8).reshape(S_tot, D)


def _ffn_kernel(te_ref, tv_ref, xs_ref, w1_ref, b1_ref, w2_ref, b2_ref,
                o_ref, acc_ref, w1b_ref, w2b_ref, *, FB, M):
    f = pl.program_id(0)
    t = pl.program_id(1)

    changed = jnp.logical_or(t == 0,
                             te_ref[t] != te_ref[jnp.maximum(t - 1, 0)])

    @pl.when(changed)
    def _cvt():
        w1b_ref[...] = w1_ref[0].astype(jnp.bfloat16)
        w2b_ref[...] = w2_ref[0].astype(jnp.bfloat16)

    @pl.when(tv_ref[t] == 1)
    def _():
        xb = xs_ref[...].astype(jnp.bfloat16)
        h = jnp.dot(xb, w1b_ref[...],
                    preferred_element_type=jnp.float32) + b1_ref[0]
        h = jnp.maximum(h, 0.0).astype(jnp.bfloat16)
        part = jnp.dot(h, w2b_ref[...],
                       preferred_element_type=jnp.float32)

        if FB == 1:
            o_ref[...] = part + b2_ref[0]
        else:
            @pl.when(f == 0)
            def _first():
                acc_ref[pl.ds(t * M, M), :] = part.astype(jnp.bfloat16)

            @pl.when(jnp.logical_and(f > 0, f < FB - 1))
            def _rest():
                acc_ref[pl.ds(t * M, M), :] = (
                    acc_ref[pl.ds(t * M, M), :].astype(jnp.float32) + part
                ).astype(jnp.bfloat16)

            @pl.when(f == FB - 1)
            def _last():
                o_ref[...] = (acc_ref[pl.ds(t * M, M), :].astype(jnp.float32)
                              + part + b2_ref[0])


def _grouped_ffn(Xs, W1, b1, W2, b2, te, tv, *, CAP, D, E, F, M, Fb):
    T = CAP // M
    FB = F // Fb

    grid_spec = pltpu.PrefetchScalarGridSpec(
        num_scalar_prefetch=2,
        grid=(FB, T),
        in_specs=[
            pl.BlockSpec((M, D), lambda f, t, te, tv: (t, 0)),
            pl.BlockSpec((1, D, Fb), lambda f, t, te, tv: (te[t], 0, f)),
            pl.BlockSpec((1, 1, Fb),
                         lambda f, t, te, tv: (te[t] * FB + f, 0, 0)),
            pl.BlockSpec((1, Fb, D), lambda f, t, te, tv: (te[t], f, 0)),
            pl.BlockSpec((1, 1, D), lambda f, t, te, tv: (te[t], 0, 0)),
        ],
        out_specs=pl.BlockSpec((M, D), lambda f, t, te, tv: (t, 0)),
        scratch_shapes=[
            pltpu.VMEM((CAP, D), jnp.bfloat16),
            pltpu.VMEM((D, Fb), jnp.bfloat16),
            pltpu.VMEM((Fb, D), jnp.bfloat16),
        ],
    )
    return pl.pallas_call(
        functools.partial(_ffn_kernel, FB=FB, M=M),
        grid_spec=grid_spec,
        out_shape=jax.ShapeDtypeStruct((CAP, D), jnp.float32),
    )(te, tv, Xs, W1, b1.reshape(E * FB, 1, Fb), W2, b2.reshape(E, 1, D))


def _ln_kernel(x_ref, g_ref, sc_ref, gm_ref, bt_ref, o_ref):
    s = sc_ref[...]
    h2 = (x_ref[...] + g_ref[:, 0, :] * s[:, 0:1] + g_ref[:, 1, :] * s[:, 1:2])
    mu = jnp.mean(h2, axis=-1, keepdims=True)
    d = h2 - mu
    var = jnp.mean(d * d, axis=-1, keepdims=True)
    o_ref[...] = d * jax.lax.rsqrt(var + _EPS) * gm_ref[...] + bt_ref[...]


def _combine_ln(xf, g, sc2, gamma, beta, *, N, D, M):
    T = N // M
    return pl.pallas_call(
        _ln_kernel,
        grid=(T,),
        in_specs=[
            pl.BlockSpec((M, D), lambda t: (t, 0)),
            pl.BlockSpec((M, 2, D), lambda t: (t, 0, 0)),
            pl.BlockSpec((M, 2), lambda t: (t, 0)),
            pl.BlockSpec((1, D), lambda t: (0, 0)),
            pl.BlockSpec((1, D), lambda t: (0, 0)),
        ],
        out_specs=pl.BlockSpec((M, D), lambda t: (t, 0)),
        out_shape=jax.ShapeDtypeStruct((N, D), jnp.float32),
    )(xf, g, sc2, gamma.reshape(1, D), beta.reshape(1, D))


def kernel(x, Wr, br, W1, b1, W2, b2, gamma, beta):
    B, S, D = x.shape
    E = Wr.shape[1]
    F = W1.shape[2]
    N = B * S
    K = 2
    M = min(512, N)
    Fb = min(1024, F)
    CAP = K * N + E * M
    T = CAP // M

    xf = x.reshape(N, D)

    idx2, sc2 = _router(xf, Wr, br, N=N, D=D, E=E, M=min(512, N))

    dest, aend = _dispatch_meta(idx2, N=N, E=E, K=K, Mal=M)

    tid = jnp.arange(T, dtype=jnp.int32).astype(jnp.float32) * M
    te = jnp.minimum(jnp.sum((tid[:, None] >= aend[None, :]), axis=1),
                     E - 1).astype(jnp.int32)
    tv = (tid < aend[E - 1]).astype(jnp.int32)

    C = D // _LANES
    cj = jnp.arange(C, dtype=jnp.int32)[None, :]
    d2 = dest.reshape(N, K)
    de8 = (d2[:, 0:1] * C + cj).reshape(1, N * C)
    do8 = (d2[:, 1:2] * C + cj).reshape(1, N * C)
    dest8 = (dest[:, None] * C + cj).reshape(1, K * N * C)

    Xs = _sc_dispatch(xf, de8, do8, N=N, D=D, CAP=CAP)
    ys = _grouped_ffn(Xs, W1, b1, W2, b2, te, tv,
                      CAP=CAP, D=D, E=E, F=F, M=M, Fb=Fb)
    g = _sc_combine(ys, dest8, S_tot=K * N, D=D)
    y = _combine_ln(xf, g.reshape(N, K, D), sc2, gamma, beta,
                    N=N, D=D, M=min(512, N))
    return y.reshape(B, S, D)


# speedup vs baseline: 1.1121x; 1.1121x over previous
import functools

import jax
import jax.numpy as jnp
from jax.experimental import pallas as pl
from jax.experimental.pallas import tpu as pltpu
from jax.experimental.pallas import tpu_sc as plsc

_EPS = 1e-5
_LANES = 128


def _router_kernel(x_ref, wr_ref, br_ref, oi_ref, os_ref, *, E, M):
    lane = jax.lax.broadcasted_iota(jnp.int32, (M, _LANES), 1)
    xb = x_ref[...].astype(jnp.bfloat16)
    logits = jnp.dot(xb, wr_ref[...].astype(jnp.bfloat16),
                     preferred_element_type=jnp.float32) + br_ref[...]
    logits = jnp.where(lane < E, logits, -jnp.inf)
    big = jnp.int32(_LANES + 1)
    m1 = jnp.max(logits, axis=-1, keepdims=True)
    i1 = jnp.min(jnp.where(logits == m1, lane, big), axis=-1, keepdims=True)
    oh1 = lane == i1
    l2 = jnp.where(oh1, -jnp.inf, logits)
    m2 = jnp.max(l2, axis=-1, keepdims=True)
    i2 = jnp.min(jnp.where(l2 == m2, lane, big), axis=-1, keepdims=True)
    oh2 = lane == i2
    p = jnp.exp(logits - m1)
    p = jnp.where(lane < E, p, 0.0)
    p = p / jnp.sum(p, axis=-1, keepdims=True)
    s1 = jnp.sum(jnp.where(oh1, p, 0.0), axis=-1, keepdims=True)
    s2 = jnp.sum(jnp.where(oh2, p, 0.0), axis=-1, keepdims=True)
    oi_ref[...] = jnp.concatenate([i1, i2], axis=1)
    os_ref[...] = jnp.concatenate([s1, s2], axis=1)


def _router(xf, Wr, br, *, N, D, E, M):
    T = N // M
    wr_p = jnp.zeros((D, _LANES), Wr.dtype).at[:, :E].set(Wr)
    br_p = jnp.zeros((1, _LANES), br.dtype).at[0, :E].set(br)
    return pl.pallas_call(
        functools.partial(_router_kernel, E=E, M=M),
        grid=(T,),
        in_specs=[
            pl.BlockSpec((M, D), lambda t: (t, 0)),
            pl.BlockSpec((D, _LANES), lambda t: (0, 0)),
            pl.BlockSpec((1, _LANES), lambda t: (0, 0)),
        ],
        out_specs=[
            pl.BlockSpec((M, 2), lambda t: (t, 0)),
            pl.BlockSpec((M, 2), lambda t: (t, 0)),
        ],
        out_shape=[
            jax.ShapeDtypeStruct((N, 2), jnp.int32),
            jax.ShapeDtypeStruct((N, 2), jnp.float32),
        ],
    )(xf, wr_p, br_p)


def _meta_kernel(idx_ref, dest_ref, aend_ref, l_scr, rank_scr, cnt_scr,
                 ast_scr, *, E, Ms, Mal):
    d = pl.program_id(0)
    t = pl.program_id(1)
    lane = jax.lax.broadcasted_iota(jnp.int32, (Ms, _LANES), 1)
    eid = idx_ref[0]
    oh = lane == eid
    ohf = oh.astype(jnp.bfloat16)

    @pl.when(jnp.logical_and(d == 0, t == 0))
    def _init():
        r = jax.lax.broadcasted_iota(jnp.int32, (Ms, Ms), 0)
        c = jax.lax.broadcasted_iota(jnp.int32, (Ms, Ms), 1)
        l_scr[...] = (r > c).astype(jnp.bfloat16)
        cnt_scr[...] = jnp.zeros_like(cnt_scr)

    @pl.when(d == 0)
    def _pass_a():
        exc = jnp.dot(l_scr[...], ohf, preferred_element_type=jnp.float32)
        intra = jnp.sum(jnp.where(oh, exc, 0.0), axis=1, keepdims=True)
        base = jnp.sum(jnp.where(oh, cnt_scr[...], 0.0), axis=1, keepdims=True)
        rank_scr[pl.ds(t * Ms, Ms), :] = base + intra
        cnt_scr[...] += jnp.sum(ohf.astype(jnp.float32), axis=0, keepdims=True)

    @pl.when(jnp.logical_and(d == 1, t == 0))
    def _offsets():
        counts = cnt_scr[...]
        sizes = jnp.ceil(counts / Mal) * Mal
        r2 = jax.lax.broadcasted_iota(jnp.int32, (_LANES, _LANES), 0)
        c2 = jax.lax.broadcasted_iota(jnp.int32, (_LANES, _LANES), 1)
        lt = (r2 <= c2).astype(jnp.bfloat16)
        aend = jnp.dot(sizes.astype(jnp.bfloat16), lt,
                       preferred_element_type=jnp.float32)
        ast_scr[...] = aend - sizes
        aend_ref[...] = aend

    @pl.when(d == 1)
    def _pass_b():
        base = jnp.sum(jnp.where(oh, ast_scr[...], 0.0), axis=1, keepdims=True)
        dest = base + rank_scr[pl.ds(t * Ms, Ms), :]
        dest_ref[0] = dest.astype(jnp.int32)


def _dispatch_meta(idx2, *, N, E, K, Mal):
    S_tot = K * N
    Ms = 1024
    T = S_tot // Ms
    idx_r = idx2.reshape(T, Ms, 1)
    dest, aend = pl.pallas_call(
        functools.partial(_meta_kernel, E=E, Ms=Ms, Mal=Mal),
        grid=(2, T),
        in_specs=[pl.BlockSpec((1, Ms, 1), lambda d, t: (t, 0, 0))],
        out_specs=[
            pl.BlockSpec((1, Ms, 1), lambda d, t: (t, 0, 0)),
            pl.BlockSpec((1, _LANES), lambda d, t: (0, 0)),
        ],
        out_shape=[
            jax.ShapeDtypeStruct((T, Ms, 1), jnp.int32),
            jax.ShapeDtypeStruct((1, _LANES), jnp.float32),
        ],
        scratch_shapes=[
            pltpu.VMEM((Ms, Ms), jnp.bfloat16),
            pltpu.VMEM((S_tot, 1), jnp.float32),
            pltpu.VMEM((1, _LANES), jnp.float32),
            pltpu.VMEM((1, _LANES), jnp.float32),
        ],
    )(idx_r)
    return dest.reshape(S_tot), aend[0, :E]


_SC_MESH = None


def _sc_mesh():
    global _SC_MESH
    if _SC_MESH is None:
        _SC_MESH = plsc.VectorSubcoreMesh(core_axis_name="c",
                                          subcore_axis_name="s")
    return _SC_MESH


_SC_W = 128


def _sc_dispatch(xf, de, do, *, N, D, CAP, W=_SC_W):
    C = D // _LANES
    x8 = xf.reshape(N * C, _LANES)
    R = N * C

    @functools.partial(
        pl.kernel,
        out_type=jax.ShapeDtypeStruct((CAP * C, _LANES), jnp.float32),
        mesh=_sc_mesh(),
    )
    def k(x_hbm, ie_hbm, io_hbm, o_hbm):
        def body(x_vmem, ie_vmem, io_vmem):
            pltpu.sync_copy(x_vmem, o_hbm.at[ie_vmem.at[0]])
            pltpu.sync_copy(x_vmem, o_hbm.at[io_vmem.at[0]])

        pltpu.emit_pipeline(
            body,
            grid=(R // W,),
            in_specs=[
                pl.BlockSpec((W, _LANES), lambda i: (i, 0)),
                pl.BlockSpec((1, W), lambda i: (0, i)),
                pl.BlockSpec((1, W), lambda i: (0, i)),
            ],
            out_specs=[],
            core_axis_name=("c", "s"),
            dimension_semantics=(pltpu.PARALLEL,),
        )(x_hbm, ie_hbm, io_hbm)

    return k(x8, de, do)


def _sc_combine(ys, dest8, *, S_tot, D, W=_SC_W):
    C = D // _LANES
    y8 = ys.reshape(ys.shape[0] * C, _LANES)
    R = S_tot * C

    @functools.partial(
        pl.kernel,
        out_type=jax.ShapeDtypeStruct((R, _LANES), jnp.float32),
        mesh=_sc_mesh(),
    )
    def k(y_hbm, i_hbm, o_hbm):
        def body(i_vmem, o_vmem):
            pltpu.sync_copy(y_hbm.at[i_vmem.at[0]], o_vmem)

        pltpu.emit_pipeline(
            body,
            grid=(R // W,),
            in_specs=[pl.BlockSpec((1, W), lambda i: (0, i))],
            out_specs=[pl.BlockSpec((W, _LANES), lambda i: (i, 0))],
            core_axis_name=("c", "s"),
            dimension_semantics=(pltpu.PARALLEL,),
        )(i_hbm, o_hbm)

    return k(y8, dest8).reshape(S_tot, D)


def _ffn_kernel(te_ref, tv_ref, xs_ref, w1_ref, b1_ref, w2_ref, b2_ref,
                o_ref, acc_ref, w1b_ref, w2b_ref, *, FB, M):
    f = pl.program_id(0)
    t = pl.program_id(1)

    changed = jnp.logical_or(t == 0,
                             te_ref[t] != te_ref[jnp.maximum(t - 1, 0)])

    @pl.when(changed)
    def _cvt():
        w1b_ref[...] = w1_ref[0].astype(jnp.bfloat16)
        w2b_ref[...] = w2_ref[0].astype(jnp.bfloat16)

    @pl.when(tv_ref[t] == 1)
    def _():
        xb = xs_ref[...].reshape(M, -1).astype(jnp.bfloat16)
        h = jnp.dot(xb, w1b_ref[...],
                    preferred_element_type=jnp.float32) + b1_ref[0]
        h = jnp.maximum(h, 0.0).astype(jnp.bfloat16)
        part = jnp.dot(h, w2b_ref[...],
                       preferred_element_type=jnp.float32)

        if FB == 1:
            o_ref[...] = part + b2_ref[0]
        else:
            @pl.when(f == 0)
            def _first():
                acc_ref[pl.ds(t * M, M), :] = part.astype(jnp.bfloat16)

            @pl.when(jnp.logical_and(f > 0, f < FB - 1))
            def _rest():
                acc_ref[pl.ds(t * M, M), :] = (
                    acc_ref[pl.ds(t * M, M), :].astype(jnp.float32) + part
                ).astype(jnp.bfloat16)

            @pl.when(f == FB - 1)
            def _last():
                o_ref[...] = (acc_ref[pl.ds(t * M, M), :].astype(jnp.float32)
                              + part + b2_ref[0])


def _grouped_ffn(Xs8, W1, b1, W2, b2, te, tv, *, CAP, D, E, F, M, Fb):
    T = CAP // M
    FB = F // Fb

    grid_spec = pltpu.PrefetchScalarGridSpec(
        num_scalar_prefetch=2,
        grid=(FB, T),
        in_specs=[
            pl.BlockSpec((M * (D // _LANES), _LANES),
                         lambda f, t, te, tv: (t, 0)),
            pl.BlockSpec((1, D, Fb), lambda f, t, te, tv: (te[t], 0, f)),
            pl.BlockSpec((1, 1, Fb),
                         lambda f, t, te, tv: (te[t] * FB + f, 0, 0)),
            pl.BlockSpec((1, Fb, D), lambda f, t, te, tv: (te[t], f, 0)),
            pl.BlockSpec((1, 1, D), lambda f, t, te, tv: (te[t], 0, 0)),
        ],
        out_specs=pl.BlockSpec(
            (M, D),
            lambda f, t, te, tv: (jax.lax.select(f == FB - 1, t, 0), 0)),
        scratch_shapes=[
            pltpu.VMEM((CAP, D), jnp.bfloat16),
            pltpu.VMEM((D, Fb), jnp.bfloat16),
            pltpu.VMEM((Fb, D), jnp.bfloat16),
        ],
    )
    return pl.pallas_call(
        functools.partial(_ffn_kernel, FB=FB, M=M),
        grid_spec=grid_spec,
        out_shape=jax.ShapeDtypeStruct((CAP, D), jnp.float32),
    )(te, tv, Xs8, W1, b1.reshape(E * FB, 1, Fb), W2, b2.reshape(E, 1, D))


def _ln_kernel(x_ref, g_ref, sc_ref, gm_ref, bt_ref, o_ref):
    s = sc_ref[...]
    h2 = (x_ref[...] + g_ref[:, 0, :] * s[:, 0:1] + g_ref[:, 1, :] * s[:, 1:2])
    mu = jnp.mean(h2, axis=-1, keepdims=True)
    d = h2 - mu
    var = jnp.mean(d * d, axis=-1, keepdims=True)
    o_ref[...] = d * jax.lax.rsqrt(var + _EPS) * gm_ref[...] + bt_ref[...]


def _combine_ln(xf, g, sc2, gamma, beta, *, N, D, M):
    T = N // M
    return pl.pallas_call(
        _ln_kernel,
        grid=(T,),
        in_specs=[
            pl.BlockSpec((M, D), lambda t: (t, 0)),
            pl.BlockSpec((M, 2, D), lambda t: (t, 0, 0)),
            pl.BlockSpec((M, 2), lambda t: (t, 0)),
            pl.BlockSpec((1, D), lambda t: (0, 0)),
            pl.BlockSpec((1, D), lambda t: (0, 0)),
        ],
        out_specs=pl.BlockSpec((M, D), lambda t: (t, 0)),
        out_shape=jax.ShapeDtypeStruct((N, D), jnp.float32),
    )(xf, g, sc2, gamma.reshape(1, D), beta.reshape(1, D))


def kernel(x, Wr, br, W1, b1, W2, b2, gamma, beta):
    B, S, D = x.shape
    E = Wr.shape[1]
    F = W1.shape[2]
    N = B * S
    K = 2
    M = min(256, N)
    Fb = min(1024, F)
    CAP = K * N + E * M
    T = CAP // M

    xf = x.reshape(N, D)

    idx2, sc2 = _router(xf, Wr, br, N=N, D=D, E=E, M=min(512, N))

    dest, aend = _dispatch_meta(idx2, N=N, E=E, K=K, Mal=M)

    tid = jnp.arange(T, dtype=jnp.int32).astype(jnp.float32) * M
    te = jnp.minimum(jnp.sum((tid[:, None] >= aend[None, :]), axis=1),
                     E - 1).astype(jnp.int32)
    tv = (tid < aend[E - 1]).astype(jnp.int32)

    C = D // _LANES
    cj = jnp.arange(C, dtype=jnp.int32)[None, :]
    d2 = dest.reshape(N, K)
    de8 = (d2[:, 0:1] * C + cj).reshape(1, N * C)
    do8 = (d2[:, 1:2] * C + cj).reshape(1, N * C)
    dest8 = (dest[:, None] * C + cj).reshape(1, K * N * C)

    Xs8 = _sc_dispatch(xf, de8, do8, N=N, D=D, CAP=CAP)
    ys = _grouped_ffn(Xs8, W1, b1, W2, b2, te, tv,
                      CAP=CAP, D=D, E=E, F=F, M=M, Fb=Fb)
    g = _sc_combine(ys, dest8, S_tot=K * N, D=D)
    y = _combine_ln(xf, g.reshape(N, K, D), sc2, gamma, beta,
                    N=N, D=D, M=min(512, N))
    return y.reshape(B, S, D)


# speedup vs baseline: 1.2802x; 1.1512x over previous
import functools

import jax
import jax.numpy as jnp
from jax.experimental import pallas as pl
from jax.experimental.pallas import tpu as pltpu
from jax.experimental.pallas import tpu_sc as plsc

_EPS = 1e-5
_LANES = 128


def _router_kernel(x_ref, wr_ref, br_ref, oi_ref, os_ref, *, E, M):
    lane = jax.lax.broadcasted_iota(jnp.int32, (M, _LANES), 1)
    xb = x_ref[...].astype(jnp.bfloat16)
    logits = jnp.dot(xb, wr_ref[...].astype(jnp.bfloat16),
                     preferred_element_type=jnp.float32) + br_ref[...]
    logits = jnp.where(lane < E, logits, -jnp.inf)
    big = jnp.int32(_LANES + 1)
    m1 = jnp.max(logits, axis=-1, keepdims=True)
    i1 = jnp.min(jnp.where(logits == m1, lane, big), axis=-1, keepdims=True)
    oh1 = lane == i1
    l2 = jnp.where(oh1, -jnp.inf, logits)
    m2 = jnp.max(l2, axis=-1, keepdims=True)
    i2 = jnp.min(jnp.where(l2 == m2, lane, big), axis=-1, keepdims=True)
    oh2 = lane == i2
    p = jnp.exp(logits - m1)
    p = jnp.where(lane < E, p, 0.0)
    p = p / jnp.sum(p, axis=-1, keepdims=True)
    s1 = jnp.sum(jnp.where(oh1, p, 0.0), axis=-1, keepdims=True)
    s2 = jnp.sum(jnp.where(oh2, p, 0.0), axis=-1, keepdims=True)
    oi_ref[...] = jnp.concatenate([i1, i2], axis=1)
    os_ref[...] = jnp.concatenate([s1, s2], axis=1)


def _router(xf, Wr, br, *, N, D, E, M):
    T = N // M
    wr_p = jnp.zeros((D, _LANES), Wr.dtype).at[:, :E].set(Wr)
    br_p = jnp.zeros((1, _LANES), br.dtype).at[0, :E].set(br)
    return pl.pallas_call(
        functools.partial(_router_kernel, E=E, M=M),
        grid=(T,),
        in_specs=[
            pl.BlockSpec((M, D), lambda t: (t, 0)),
            pl.BlockSpec((D, _LANES), lambda t: (0, 0)),
            pl.BlockSpec((1, _LANES), lambda t: (0, 0)),
        ],
        out_specs=[
            pl.BlockSpec((M, 2), lambda t: (t, 0)),
            pl.BlockSpec((M, 2), lambda t: (t, 0)),
        ],
        out_shape=[
            jax.ShapeDtypeStruct((N, 2), jnp.int32),
            jax.ShapeDtypeStruct((N, 2), jnp.float32),
        ],
    )(xf, wr_p, br_p)


def _meta_kernel(idx_ref, dest_ref, aend_ref, l_scr, rank_scr, cnt_scr,
                 ast_scr, *, E, Ms, Mal):
    d = pl.program_id(0)
    t = pl.program_id(1)
    lane = jax.lax.broadcasted_iota(jnp.int32, (Ms, _LANES), 1)
    eid = idx_ref[0]
    oh = lane == eid
    ohf = oh.astype(jnp.bfloat16)

    @pl.when(jnp.logical_and(d == 0, t == 0))
    def _init():
        r = jax.lax.broadcasted_iota(jnp.int32, (Ms, Ms), 0)
        c = jax.lax.broadcasted_iota(jnp.int32, (Ms, Ms), 1)
        l_scr[...] = (r > c).astype(jnp.bfloat16)
        cnt_scr[...] = jnp.zeros_like(cnt_scr)

    @pl.when(d == 0)
    def _pass_a():
        exc = jnp.dot(l_scr[...], ohf, preferred_element_type=jnp.float32)
        intra = jnp.sum(jnp.where(oh, exc, 0.0), axis=1, keepdims=True)
        base = jnp.sum(jnp.where(oh, cnt_scr[...], 0.0), axis=1, keepdims=True)
        rank_scr[pl.ds(t * Ms, Ms), :] = base + intra
        cnt_scr[...] += jnp.sum(ohf.astype(jnp.float32), axis=0, keepdims=True)

    @pl.when(jnp.logical_and(d == 1, t == 0))
    def _offsets():
        counts = cnt_scr[...]
        sizes = jnp.ceil(counts / Mal) * Mal
        r2 = jax.lax.broadcasted_iota(jnp.int32, (_LANES, _LANES), 0)
        c2 = jax.lax.broadcasted_iota(jnp.int32, (_LANES, _LANES), 1)
        lt = (r2 <= c2).astype(jnp.bfloat16)
        aend = jnp.dot(sizes.astype(jnp.bfloat16), lt,
                       preferred_element_type=jnp.float32)
        ast_scr[...] = aend - sizes
        aend_ref[...] = aend

    @pl.when(d == 1)
    def _pass_b():
        base = jnp.sum(jnp.where(oh, ast_scr[...], 0.0), axis=1, keepdims=True)
        dest = base + rank_scr[pl.ds(t * Ms, Ms), :]
        dest_ref[0] = dest.astype(jnp.int32)


def _dispatch_meta(idx2, *, N, E, K, Mal):
    S_tot = K * N
    Ms = 1024
    T = S_tot // Ms
    idx_r = idx2.reshape(T, Ms, 1)
    dest, aend = pl.pallas_call(
        functools.partial(_meta_kernel, E=E, Ms=Ms, Mal=Mal),
        grid=(2, T),
        in_specs=[pl.BlockSpec((1, Ms, 1), lambda d, t: (t, 0, 0))],
        out_specs=[
            pl.BlockSpec((1, Ms, 1), lambda d, t: (t, 0, 0)),
            pl.BlockSpec((1, _LANES), lambda d, t: (0, 0)),
        ],
        out_shape=[
            jax.ShapeDtypeStruct((T, Ms, 1), jnp.int32),
            jax.ShapeDtypeStruct((1, _LANES), jnp.float32),
        ],
        scratch_shapes=[
            pltpu.VMEM((Ms, Ms), jnp.bfloat16),
            pltpu.VMEM((S_tot, 1), jnp.float32),
            pltpu.VMEM((1, _LANES), jnp.float32),
            pltpu.VMEM((1, _LANES), jnp.float32),
        ],
    )(idx_r)
    return dest.reshape(S_tot), aend[0, :E]


_SC_MESH = None


def _sc_mesh():
    global _SC_MESH
    if _SC_MESH is None:
        _SC_MESH = plsc.VectorSubcoreMesh(core_axis_name="c",
                                          subcore_axis_name="s")
    return _SC_MESH


_SC_W = 128


def _sc_dispatch(xf, de, do, *, N, D, CAP, W=_SC_W):
    C = D // _LANES
    x8 = xf.reshape(N * C, _LANES)
    R = N * C

    @functools.partial(
        pl.kernel,
        out_type=jax.ShapeDtypeStruct((CAP * C, _LANES), jnp.float32),
        mesh=_sc_mesh(),
    )
    def k(x_hbm, ie_hbm, io_hbm, o_hbm):
        def body(x_vmem, ie_vmem, io_vmem):
            pltpu.sync_copy(x_vmem, o_hbm.at[ie_vmem.at[0]])
            pltpu.sync_copy(x_vmem, o_hbm.at[io_vmem.at[0]])

        pltpu.emit_pipeline(
            body,
            grid=(R // W,),
            in_specs=[
                pl.BlockSpec((W, _LANES), lambda i: (i, 0)),
                pl.BlockSpec((1, W), lambda i: (0, i)),
                pl.BlockSpec((1, W), lambda i: (0, i)),
            ],
            out_specs=[],
            core_axis_name=("c", "s"),
            dimension_semantics=(pltpu.PARALLEL,),
        )(x_hbm, ie_hbm, io_hbm)

    return k(x8, de, do)


def _sc_combine(y8, dest8, *, S_tot, D, W=_SC_W):
    C = D // _LANES
    R = S_tot * C

    @functools.partial(
        pl.kernel,
        out_type=jax.ShapeDtypeStruct((R, _LANES), y8.dtype),
        mesh=_sc_mesh(),
    )
    def k(y_hbm, i_hbm, o_hbm):
        def body(i_vmem, o_vmem):
            pltpu.sync_copy(y_hbm.at[i_vmem.at[0]], o_vmem)

        pltpu.emit_pipeline(
            body,
            grid=(R // W,),
            in_specs=[pl.BlockSpec((1, W), lambda i: (0, i))],
            out_specs=[pl.BlockSpec((W, _LANES), lambda i: (i, 0))],
            core_axis_name=("c", "s"),
            dimension_semantics=(pltpu.PARALLEL,),
        )(i_hbm, o_hbm)

    return k(y8, dest8)


def _ffn_kernel(te_ref, tv_ref, xs_ref, w1_ref, b1_ref, w2_ref, b2_ref,
                o_ref, acc_ref, w1b_ref, w2b_ref, *, FB, M):
    f = pl.program_id(0)
    t = pl.program_id(1)

    changed = jnp.logical_or(t == 0,
                             te_ref[t] != te_ref[jnp.maximum(t - 1, 0)])

    @pl.when(changed)
    def _cvt():
        w1b_ref[...] = w1_ref[0].astype(jnp.bfloat16)
        w2b_ref[...] = w2_ref[0].astype(jnp.bfloat16)

    @pl.when(tv_ref[t] == 1)
    def _():
        xb = xs_ref[...].reshape(M, -1).astype(jnp.bfloat16)
        h = jnp.dot(xb, w1b_ref[...],
                    preferred_element_type=jnp.float32) + b1_ref[0]
        h = jnp.maximum(h, 0.0).astype(jnp.bfloat16)
        part = jnp.dot(h, w2b_ref[...],
                       preferred_element_type=jnp.float32)

        if FB == 1:
            y1 = part + b2_ref[0]
            o_ref[...] = y1.reshape(o_ref.shape)
        else:
            @pl.when(f == 0)
            def _first():
                acc_ref[pl.ds(t * M, M), :] = part.astype(jnp.bfloat16)

            @pl.when(jnp.logical_and(f > 0, f < FB - 1))
            def _rest():
                acc_ref[pl.ds(t * M, M), :] = (
                    acc_ref[pl.ds(t * M, M), :].astype(jnp.float32) + part
                ).astype(jnp.bfloat16)

            @pl.when(f == FB - 1)
            def _last():
                y = (acc_ref[pl.ds(t * M, M), :].astype(jnp.float32)
                     + part + b2_ref[0])
                o_ref[...] = y.reshape(o_ref.shape)


def _grouped_ffn(Xs8, W1, b1, W2, b2, te, tv, *, CAP, D, E, F, M, Fb):
    T = CAP // M
    FB = F // Fb

    grid_spec = pltpu.PrefetchScalarGridSpec(
        num_scalar_prefetch=2,
        grid=(FB, T),
        in_specs=[
            pl.BlockSpec((M * (D // _LANES), _LANES),
                         lambda f, t, te, tv: (t, 0)),
            pl.BlockSpec((1, D, Fb), lambda f, t, te, tv: (te[t], 0, f)),
            pl.BlockSpec((1, 1, Fb),
                         lambda f, t, te, tv: (te[t] * FB + f, 0, 0)),
            pl.BlockSpec((1, Fb, D), lambda f, t, te, tv: (te[t], f, 0)),
            pl.BlockSpec((1, 1, D), lambda f, t, te, tv: (te[t], 0, 0)),
        ],
        out_specs=pl.BlockSpec(
            (M * (D // _LANES), _LANES),
            lambda f, t, te, tv: (jax.lax.select(f == FB - 1, t, 0), 0)),
        scratch_shapes=[
            pltpu.VMEM((CAP, D), jnp.bfloat16),
            pltpu.VMEM((D, Fb), jnp.bfloat16),
            pltpu.VMEM((Fb, D), jnp.bfloat16),
        ],
    )
    return pl.pallas_call(
        functools.partial(_ffn_kernel, FB=FB, M=M),
        grid_spec=grid_spec,
        out_shape=jax.ShapeDtypeStruct((CAP * (D // _LANES), _LANES),
                                       jnp.float32),
    )(te, tv, Xs8, W1, b1.reshape(E * FB, 1, Fb), W2, b2.reshape(E, 1, D))


def _ln_kernel(x_ref, g_ref, sc_ref, gm_ref, bt_ref, o_ref, *, M, K, D):
    s = sc_ref[...]
    g = g_ref[...].astype(jnp.float32).reshape(M, K, D)
    h2 = (x_ref[...] + g[:, 0, :] * s[:, 0:1] + g[:, 1, :] * s[:, 1:2])
    mu = jnp.mean(h2, axis=-1, keepdims=True)
    d = h2 - mu
    var = jnp.mean(d * d, axis=-1, keepdims=True)
    o_ref[...] = d * jax.lax.rsqrt(var + _EPS) * gm_ref[...] + bt_ref[...]


def _combine_ln(xf, g8, sc2, gamma, beta, *, N, D, K, M):
    T = N // M
    C = D // _LANES
    return pl.pallas_call(
        functools.partial(_ln_kernel, M=M, K=K, D=D),
        grid=(T,),
        in_specs=[
            pl.BlockSpec((M, D), lambda t: (t, 0)),
            pl.BlockSpec((M * K * C, _LANES), lambda t: (t, 0)),
            pl.BlockSpec((M, 2), lambda t: (t, 0)),
            pl.BlockSpec((1, D), lambda t: (0, 0)),
            pl.BlockSpec((1, D), lambda t: (0, 0)),
        ],
        out_specs=pl.BlockSpec((M, D), lambda t: (t, 0)),
        out_shape=jax.ShapeDtypeStruct((N, D), jnp.float32),
    )(xf, g8, sc2, gamma.reshape(1, D), beta.reshape(1, D))


def kernel(x, Wr, br, W1, b1, W2, b2, gamma, beta):
    B, S, D = x.shape
    E = Wr.shape[1]
    F = W1.shape[2]
    N = B * S
    K = 2
    M = min(256, N)
    Fb = min(1024, F)
    CAP = K * N + E * M
    T = CAP // M

    xf = x.reshape(N, D)

    idx2, sc2 = _router(xf, Wr, br, N=N, D=D, E=E, M=min(512, N))

    dest, aend = _dispatch_meta(idx2, N=N, E=E, K=K, Mal=M)

    tid = jnp.arange(T, dtype=jnp.int32).astype(jnp.float32) * M
    te = jnp.minimum(jnp.sum((tid[:, None] >= aend[None, :]), axis=1),
                     E - 1).astype(jnp.int32)
    tv = (tid < aend[E - 1]).astype(jnp.int32)

    C = D // _LANES
    cj = jnp.arange(C, dtype=jnp.int32)[None, :]
    d2 = dest.reshape(N, K)
    de8 = (d2[:, 0:1] * C + cj).reshape(1, N * C)
    do8 = (d2[:, 1:2] * C + cj).reshape(1, N * C)
    dest8 = (dest[:, None] * C + cj).reshape(1, K * N * C)

    Xs8 = _sc_dispatch(xf, de8, do8, N=N, D=D, CAP=CAP)
    ys = _grouped_ffn(Xs8, W1, b1, W2, b2, te, tv,
                      CAP=CAP, D=D, E=E, F=F, M=M, Fb=Fb)
    g8 = _sc_combine(ys, dest8, S_tot=K * N, D=D)
    y = _combine_ln(xf, g8, sc2, gamma, beta,
                    N=N, D=D, K=K, M=min(512, N))
    return y.reshape(B, S, D)
